# Initial kernel scaffold; baseline (speedup 1.0000x reference)
import jax
import jax.numpy as jnp
from jax.experimental import pallas as pl


def kernel(x, edge_index, W1, W2):
    raise NotImplementedError("write your pallas kernel here")


# speedup vs baseline: 10.4078x; 10.4078x over previous
import functools

import jax
import jax.numpy as jnp
from jax import lax
from jax.experimental import pallas as pl
from jax.experimental.pallas import tpu as pltpu
from jax.experimental.pallas import tpu_sc as plsc

NN = 10000
NP = 10240
EE = 320000
DD = 128
NC = 2
NS = 16
NW = NC * NS
ET = EE // NW
C = 80
NCHUNK = ET // C
RT = NP // NS
ZR = 128
ZR2 = 64

_mesh = plsc.VectorSubcoreMesh(core_axis_name="c", subcore_axis_name="s")


def _zero_fill(zer_v, rows, width):
    @pl.loop(0, rows)
    def _(i):
        @pl.loop(0, width // 16)
        def _(j):
            zer_v[i, pl.ds(j * 16, 16)] = jnp.zeros((16,), jnp.float32)


@functools.partial(
    pl.kernel,
    out_type=jax.ShapeDtypeStruct((NC, NP, DD), jnp.float32),
    mesh=_mesh,
    scratch_types=[
        pltpu.VMEM((C,), jnp.int32),
        pltpu.VMEM((C,), jnp.int32),
        pltpu.VMEM((C, DD), jnp.float32),
        pltpu.VMEM((ZR2, DD), jnp.float32),
        pltpu.VMEM_SHARED((NP, DD), jnp.float32),
        pltpu.SemaphoreType.DMA,
    ],
)
def _agg_partials(g_hbm, src_hbm, dst_hbm, out_hbm,
                  sidx_v, didx_v, rows_v, zer_v, acc_sh, sem):
    cid = lax.axis_index("c")
    sid = lax.axis_index("s")
    wid = sid * NC + cid

    _zero_fill(zer_v, ZR2, DD)

    @pl.loop(0, RT // ZR2)
    def _(k):
        pltpu.sync_copy(zer_v, acc_sh.at[pl.ds(sid * RT + k * ZR2, ZR2)])
    plsc.subcore_barrier()

    @pl.loop(0, NCHUNK)
    def _(ci):
        pltpu.sync_copy(src_hbm.at[pl.ds(wid * ET + ci * C, C)], sidx_v)
        pltpu.sync_copy(dst_hbm.at[pl.ds(wid * ET + ci * C, C)], didx_v)
        pltpu.async_copy(g_hbm.at[sidx_v], rows_v, sem).wait()
        pltpu.sync_copy(rows_v, acc_sh.at[didx_v], add=True)

    plsc.subcore_barrier()
    pltpu.sync_copy(acc_sh.at[pl.ds(sid * RT, RT)],
                    out_hbm.at[cid, pl.ds(sid * RT, RT)])


_BN = 2000


def _mm_body(x_ref, w_ref, o_ref):
    o_ref[...] = jnp.dot(x_ref[...], w_ref[...],
                         preferred_element_type=jnp.float32)


def _matmul(x, w):
    return pl.pallas_call(
        _mm_body,
        grid=(NN // _BN,),
        in_specs=[
            pl.BlockSpec((_BN, DD), lambda i: (i, 0)),
            pl.BlockSpec((DD, DD), lambda i: (0, 0)),
        ],
        out_specs=pl.BlockSpec((_BN, DD), lambda i: (i, 0)),
        out_shape=jax.ShapeDtypeStruct((NN, DD), jnp.float32),
    )(x, w)


def _scale_body(degp_ref, h_ref, dis_ref, g_ref):
    deg = degp_ref[0, :, :16] + degp_ref[1, :, :16] + 1.0
    dis = lax.rsqrt(deg)
    dis_ref[...] = dis
    g_ref[...] = dis[:, :1] * h_ref[...]


def _dis_and_scale(degp, h):
    return pl.pallas_call(
        _scale_body,
        grid=(NN // _BN,),
        in_specs=[
            pl.BlockSpec((NC, _BN, DD), lambda i: (0, i, 0)),
            pl.BlockSpec((_BN, DD), lambda i: (i, 0)),
        ],
        out_specs=[
            pl.BlockSpec((_BN, 16), lambda i: (i, 0)),
            pl.BlockSpec((_BN, DD), lambda i: (i, 0)),
        ],
        out_shape=[
            jax.ShapeDtypeStruct((NN, 16), jnp.float32),
            jax.ShapeDtypeStruct((NN, DD), jnp.float32),
        ],
    )(degp, h)


def _mid_body(sp_ref, g_ref, dis_ref, w_ref, o_ref):
    dis = dis_ref[...][:, :1]
    z = jnp.maximum(dis * (sp_ref[0] + sp_ref[1] + g_ref[...]), 0.0)
    o_ref[...] = dis * jnp.dot(z, w_ref[...],
                               preferred_element_type=jnp.float32)


def _mid_layer(sp, g, dis, w):
    return pl.pallas_call(
        _mid_body,
        grid=(NN // _BN,),
        in_specs=[
            pl.BlockSpec((NC, _BN, DD), lambda i: (0, i, 0)),
            pl.BlockSpec((_BN, DD), lambda i: (i, 0)),
            pl.BlockSpec((_BN, 16), lambda i: (i, 0)),
            pl.BlockSpec((DD, DD), lambda i: (0, 0)),
        ],
        out_specs=pl.BlockSpec((_BN, DD), lambda i: (i, 0)),
        out_shape=jax.ShapeDtypeStruct((NN, DD), jnp.float32),
    )(sp, g, dis, w)


def _final_body(sp_ref, g_ref, dis_ref, o_ref):
    dis = dis_ref[...][:, :1]
    o_ref[...] = dis * (sp_ref[0] + sp_ref[1] + g_ref[...])


def _final_layer(sp, g, dis):
    return pl.pallas_call(
        _final_body,
        grid=(NN // _BN,),
        in_specs=[
            pl.BlockSpec((NC, _BN, DD), lambda i: (0, i, 0)),
            pl.BlockSpec((_BN, DD), lambda i: (i, 0)),
            pl.BlockSpec((_BN, 16), lambda i: (i, 0)),
        ],
        out_specs=pl.BlockSpec((_BN, DD), lambda i: (i, 0)),
        out_shape=jax.ShapeDtypeStruct((NN, DD), jnp.float32),
    )(sp, g, dis)


def kernel(x, edge_index, W1, W2):
    src3 = edge_index[0]
    dst3 = edge_index[1]

    ones = jnp.ones((NN, DD), jnp.float32)
    degp = _agg_partials(ones, src3, dst3)
    h1 = _matmul(x, W1)
    dis, g1 = _dis_and_scale(degp, h1)
    s1p = _agg_partials(g1, src3, dst3)
    g2 = _mid_layer(s1p, g1, dis, W2)
    s2p = _agg_partials(g2, src3, dst3)
    out = _final_layer(s2p, g2, dis)
    return out


# speedup vs baseline: 20.8185x; 2.0003x over previous
import functools

import jax
import jax.numpy as jnp
from jax import lax
from jax.experimental import pallas as pl
from jax.experimental.pallas import tpu as pltpu
from jax.experimental.pallas import tpu_sc as plsc

NN = 10000
NP = 10240
EE = 320000
DD = 128
NC = 2
NS = 16
NW = NC * NS
ET = EE // NW
C = 80
NCHUNK = ET // C
RT = NP // NS
ZR = 64

_mesh = plsc.VectorSubcoreMesh(core_axis_name="c", subcore_axis_name="s")


def _fill(ref, rows, width, value):
    @pl.loop(0, rows)
    def _(i):
        @pl.loop(0, width // 16)
        def _(j):
            ref[i, pl.ds(j * 16, 16)] = jnp.full((16,), value, jnp.float32)


def _zero_acc(zer_v, acc_sh, sid):
    _fill(zer_v, ZR, DD, 0.0)

    @pl.loop(0, RT // ZR)
    def _(k):
        pltpu.sync_copy(zer_v, acc_sh.at[pl.ds(sid * RT + k * ZR, ZR)])


def _readback(acc_sh, out_hbm, cid, sid):
    pltpu.sync_copy(acc_sh.at[pl.ds(sid * RT, RT)],
                    out_hbm.at[cid, pl.ds(sid * RT, RT)])


@functools.partial(
    pl.kernel,
    out_type=jax.ShapeDtypeStruct((NC, NP, DD), jnp.float32),
    mesh=_mesh,
    scratch_types=[
        pltpu.VMEM((C,), jnp.int32),
        pltpu.VMEM((C,), jnp.int32),
        pltpu.VMEM((C, DD), jnp.float32),
        pltpu.VMEM((ZR, DD), jnp.float32),
        pltpu.VMEM_SHARED((NP, DD), jnp.float32),
        pltpu.SemaphoreType.DMA,
        pltpu.SemaphoreType.DMA,
    ],
)
def _deg_partials(dst_hbm, out_hbm, dia, dib, ones_v, zer_v, acc_sh,
                  sem_a, sem_b):
    cid = lax.axis_index("c")
    sid = lax.axis_index("s")
    wid = sid * NC + cid
    base = wid * ET

    _fill(ones_v, C, DD, 1.0)
    _zero_acc(zer_v, acc_sh, sid)
    plsc.subcore_barrier()

    pltpu.sync_copy(dst_hbm.at[pl.ds(base, C)], dia)
    pltpu.async_copy(ones_v, acc_sh.at[dia], sem_a, add=True)

    @pl.loop(0, NCHUNK // 2)
    def _(k):
        ci = 2 * k
        pltpu.sync_copy(dst_hbm.at[pl.ds(base + (ci + 1) * C, C)], dib)
        pltpu.async_copy(ones_v, acc_sh.at[dib], sem_b, add=True)
        pltpu.make_async_copy(ones_v, acc_sh.at[dia], sem_a).wait()
        pltpu.sync_copy(dst_hbm.at[pl.ds(base + (ci + 2) * C, C)], dia)
        pltpu.async_copy(ones_v, acc_sh.at[dia], sem_a, add=True)
        pltpu.make_async_copy(ones_v, acc_sh.at[dib], sem_b).wait()

    pltpu.make_async_copy(ones_v, acc_sh.at[dia], sem_a).wait()
    plsc.subcore_barrier()
    _readback(acc_sh, out_hbm, cid, sid)


@functools.partial(
    pl.kernel,
    out_type=jax.ShapeDtypeStruct((NC, NP, DD), jnp.float32),
    mesh=_mesh,
    scratch_types=[
        pltpu.VMEM((C,), jnp.int32),
        pltpu.VMEM((C,), jnp.int32),
        pltpu.VMEM((C,), jnp.int32),
        pltpu.VMEM((C,), jnp.int32),
        pltpu.VMEM((C, DD), jnp.float32),
        pltpu.VMEM((C, DD), jnp.float32),
        pltpu.VMEM((ZR, DD), jnp.float32),
        pltpu.VMEM_SHARED((NP, DD), jnp.float32),
        pltpu.SemaphoreType.DMA,
        pltpu.SemaphoreType.DMA,
    ],
)
def _agg_partials(g_hbm, src_hbm, dst_hbm, out_hbm,
                  sia, sib, dia, dib, rows_a, rows_b, zer_v, acc_sh,
                  sem_a, sem_b):
    cid = lax.axis_index("c")
    sid = lax.axis_index("s")
    wid = sid * NC + cid
    base = wid * ET

    _zero_acc(zer_v, acc_sh, sid)
    plsc.subcore_barrier()

    def load_idx(buf_s, buf_d, ci):
        pltpu.sync_copy(src_hbm.at[pl.ds(base + ci * C, C)], buf_s)
        pltpu.sync_copy(dst_hbm.at[pl.ds(base + ci * C, C)], buf_d)

    load_idx(sia, dia, 0)
    pltpu.async_copy(g_hbm.at[sia], rows_a, sem_a)

    @pl.loop(0, NCHUNK // 2)
    def _(k):
        ci = 2 * k
        load_idx(sib, dib, ci + 1)
        pltpu.async_copy(g_hbm.at[sib], rows_b, sem_b)
        pltpu.make_async_copy(g_hbm.at[sia], rows_a, sem_a).wait()
        pltpu.sync_copy(rows_a, acc_sh.at[dia], add=True)
        load_idx(sia, dia, ci + 2)
        pltpu.async_copy(g_hbm.at[sia], rows_a, sem_a)
        pltpu.make_async_copy(g_hbm.at[sib], rows_b, sem_b).wait()
        pltpu.sync_copy(rows_b, acc_sh.at[dib], add=True)

    pltpu.make_async_copy(g_hbm.at[sia], rows_a, sem_a).wait()
    pltpu.sync_copy(rows_a, acc_sh.at[dia], add=True)
    plsc.subcore_barrier()
    _readback(acc_sh, out_hbm, cid, sid)


_BN = 2000


def _mm_body(x_ref, w_ref, o_ref):
    o_ref[...] = jnp.dot(x_ref[...], w_ref[...],
                         preferred_element_type=jnp.float32)


def _matmul(x, w):
    return pl.pallas_call(
        _mm_body,
        grid=(NN // _BN,),
        in_specs=[
            pl.BlockSpec((_BN, DD), lambda i: (i, 0)),
            pl.BlockSpec((DD, DD), lambda i: (0, 0)),
        ],
        out_specs=pl.BlockSpec((_BN, DD), lambda i: (i, 0)),
        out_shape=jax.ShapeDtypeStruct((NN, DD), jnp.float32),
    )(x, w)


def _scale_body(degp_ref, h_ref, dis_ref, g_ref):
    deg = degp_ref[0, :, :16] + degp_ref[1, :, :16] + 1.0
    dis = lax.rsqrt(deg)
    dis_ref[...] = dis
    g_ref[...] = dis[:, :1] * h_ref[...]


def _dis_and_scale(degp, h):
    return pl.pallas_call(
        _scale_body,
        grid=(NN // _BN,),
        in_specs=[
            pl.BlockSpec((NC, _BN, DD), lambda i: (0, i, 0)),
            pl.BlockSpec((_BN, DD), lambda i: (i, 0)),
        ],
        out_specs=[
            pl.BlockSpec((_BN, 16), lambda i: (i, 0)),
            pl.BlockSpec((_BN, DD), lambda i: (i, 0)),
        ],
        out_shape=[
            jax.ShapeDtypeStruct((NN, 16), jnp.float32),
            jax.ShapeDtypeStruct((NN, DD), jnp.float32),
        ],
    )(degp, h)


def _mid_body(sp_ref, g_ref, dis_ref, w_ref, o_ref):
    dis = dis_ref[...][:, :1]
    z = jnp.maximum(dis * (sp_ref[0] + sp_ref[1] + g_ref[...]), 0.0)
    o_ref[...] = dis * jnp.dot(z, w_ref[...],
                               preferred_element_type=jnp.float32)


def _mid_layer(sp, g, dis, w):
    return pl.pallas_call(
        _mid_body,
        grid=(NN // _BN,),
        in_specs=[
            pl.BlockSpec((NC, _BN, DD), lambda i: (0, i, 0)),
            pl.BlockSpec((_BN, DD), lambda i: (i, 0)),
            pl.BlockSpec((_BN, 16), lambda i: (i, 0)),
            pl.BlockSpec((DD, DD), lambda i: (0, 0)),
        ],
        out_specs=pl.BlockSpec((_BN, DD), lambda i: (i, 0)),
        out_shape=jax.ShapeDtypeStruct((NN, DD), jnp.float32),
    )(sp, g, dis, w)


def _final_body(sp_ref, g_ref, dis_ref, o_ref):
    dis = dis_ref[...][:, :1]
    o_ref[...] = dis * (sp_ref[0] + sp_ref[1] + g_ref[...])


def _final_layer(sp, g, dis):
    return pl.pallas_call(
        _final_body,
        grid=(NN // _BN,),
        in_specs=[
            pl.BlockSpec((NC, _BN, DD), lambda i: (0, i, 0)),
            pl.BlockSpec((_BN, DD), lambda i: (i, 0)),
            pl.BlockSpec((_BN, 16), lambda i: (i, 0)),
        ],
        out_specs=pl.BlockSpec((_BN, DD), lambda i: (i, 0)),
        out_shape=jax.ShapeDtypeStruct((NN, DD), jnp.float32),
    )(sp, g, dis)


def kernel(x, edge_index, W1, W2):
    src = edge_index[0]
    dst = edge_index[1]

    degp = _deg_partials(dst)
    h1 = _matmul(x, W1)
    dis, g1 = _dis_and_scale(degp, h1)
    s1p = _agg_partials(g1, src, dst)
    g2 = _mid_layer(s1p, g1, dis, W2)
    s2p = _agg_partials(g2, src, dst)
    out = _final_layer(s2p, g2, dis)
    return out


# speedup vs baseline: 24.3282x; 1.1686x over previous
import functools

import jax
import jax.numpy as jnp
from jax import lax
from jax.experimental import pallas as pl
from jax.experimental.pallas import tpu as pltpu
from jax.experimental.pallas import tpu_sc as plsc

NN = 10000
NP = 10240
EE = 320000
DD = 128
NC = 2
NS = 16
NW = NC * NS
ET = EE // NW
C = 80
NCHUNK = ET // C
RT = NP // NS
ZR = 64

_mesh = plsc.VectorSubcoreMesh(core_axis_name="c", subcore_axis_name="s")


def _fill(ref, rows, width, value):
    @pl.loop(0, rows)
    def _(i):
        @pl.loop(0, width // 16)
        def _(j):
            ref[i, pl.ds(j * 16, 16)] = jnp.full((16,), value, jnp.float32)


def _zero_acc(zer_v, acc_sh, sid):
    _fill(zer_v, ZR, DD, 0.0)

    @pl.loop(0, RT // ZR)
    def _(k):
        pltpu.sync_copy(zer_v, acc_sh.at[pl.ds(sid * RT + k * ZR, ZR)])


def _readback(acc_sh, out_hbm, cid, sid):
    pltpu.sync_copy(acc_sh.at[pl.ds(sid * RT, RT)],
                    out_hbm.at[cid, pl.ds(sid * RT, RT)])


@functools.partial(
    pl.kernel,
    out_type=jax.ShapeDtypeStruct((NC, NP, DD), jnp.float32),
    mesh=_mesh,
    scratch_types=[
        pltpu.VMEM((C,), jnp.int32),
        pltpu.VMEM((C,), jnp.int32),
        pltpu.VMEM((C, DD), jnp.float32),
        pltpu.VMEM((ZR, DD), jnp.float32),
        pltpu.VMEM_SHARED((NP, DD), jnp.float32),
        pltpu.SemaphoreType.DMA,
        pltpu.SemaphoreType.DMA,
    ],
)
def _deg_partials(dst_hbm, out_hbm, dia, dib, ones_v, zer_v, acc_sh,
                  sem_a, sem_b):
    cid = lax.axis_index("c")
    sid = lax.axis_index("s")
    wid = sid * NC + cid
    base = wid * ET

    _fill(ones_v, C, DD, 1.0)
    _zero_acc(zer_v, acc_sh, sid)
    plsc.subcore_barrier()

    pltpu.sync_copy(dst_hbm.at[pl.ds(base, C)], dia)
    pltpu.async_copy(ones_v, acc_sh.at[dia], sem_a, add=True)

    @pl.loop(0, NCHUNK // 2)
    def _(k):
        ci = 2 * k
        pltpu.sync_copy(dst_hbm.at[pl.ds(base + (ci + 1) * C, C)], dib)
        pltpu.async_copy(ones_v, acc_sh.at[dib], sem_b, add=True)
        pltpu.make_async_copy(ones_v, acc_sh.at[dia], sem_a).wait()
        pltpu.sync_copy(dst_hbm.at[pl.ds(base + (ci + 2) * C, C)], dia)
        pltpu.async_copy(ones_v, acc_sh.at[dia], sem_a, add=True)
        pltpu.make_async_copy(ones_v, acc_sh.at[dib], sem_b).wait()

    pltpu.make_async_copy(ones_v, acc_sh.at[dia], sem_a).wait()
    plsc.subcore_barrier()
    _readback(acc_sh, out_hbm, cid, sid)


@functools.partial(
    pl.kernel,
    out_type=jax.ShapeDtypeStruct((NC, NP, DD), jnp.float32),
    mesh=_mesh,
    scratch_types=[
        pltpu.VMEM((C,), jnp.int32),
        pltpu.VMEM((C,), jnp.int32),
        pltpu.VMEM((C,), jnp.int32),
        pltpu.VMEM((C,), jnp.int32),
        pltpu.VMEM((C,), jnp.int32),
        pltpu.VMEM((C,), jnp.int32),
        pltpu.VMEM((C, DD), jnp.float32),
        pltpu.VMEM((C, DD), jnp.float32),
        pltpu.VMEM((C, DD), jnp.float32),
        pltpu.VMEM((ZR, DD), jnp.float32),
        pltpu.VMEM_SHARED((NP, DD), jnp.float32),
        pltpu.SemaphoreType.DMA,
        pltpu.SemaphoreType.DMA,
        pltpu.SemaphoreType.DMA,
        pltpu.SemaphoreType.DMA,
        pltpu.SemaphoreType.DMA,
        pltpu.SemaphoreType.DMA,
    ],
)
def _agg_partials(g_hbm, src_hbm, dst_hbm, out_hbm,
                  si0, si1, si2, di0, di1, di2, r0, r1, r2, zer_v, acc_sh,
                  g0, g1, g2, s0, s1, s2):
    cid = lax.axis_index("c")
    sid = lax.axis_index("s")
    wid = sid * NC + cid
    base = wid * ET
    si = (si0, si1, si2)
    di = (di0, di1, di2)
    rows = (r0, r1, r2)
    gsem = (g0, g1, g2)
    ssem = (s0, s1, s2)

    _zero_acc(zer_v, acc_sh, sid)
    plsc.subcore_barrier()

    def load_idx(j, ci):
        pltpu.sync_copy(src_hbm.at[pl.ds(base + ci * C, C)], si[j])
        pltpu.sync_copy(dst_hbm.at[pl.ds(base + ci * C, C)], di[j])

    def start_gather(j):
        pltpu.async_copy(g_hbm.at[si[j]], rows[j], gsem[j])

    def wait_gather(j):
        pltpu.make_async_copy(g_hbm.at[si[j]], rows[j], gsem[j]).wait()

    def start_scatter(j):
        pltpu.async_copy(rows[j], acc_sh.at[di[j]], ssem[j], add=True)

    def wait_scatter(j):
        pltpu.make_async_copy(rows[j], acc_sh.at[di[j]], ssem[j]).wait()

    for j in (0, 1):
        load_idx(j, j)
        start_gather(j)

    @pl.loop(0, NCHUNK // 3)
    def _(k):
        for j in range(3):
            ci = 3 * k + j
            jb = (j + 2) % 3
            wait_gather(j)
            start_scatter(j)
            if j == 0:
                @pl.when(k >= 1)
                def _():
                    wait_scatter(jb)
            else:
                wait_scatter(jb)
            load_idx(jb, ci + 2)
            start_gather(jb)

    wait_gather(0)
    start_scatter(0)
    wait_gather(1)
    start_scatter(1)
    wait_scatter(2)
    wait_scatter(0)
    wait_scatter(1)
    plsc.subcore_barrier()
    _readback(acc_sh, out_hbm, cid, sid)


_BN = 10000


def _mm_body(x_ref, w_ref, o_ref):
    o_ref[...] = jnp.dot(x_ref[...], w_ref[...],
                         preferred_element_type=jnp.float32)


def _matmul(x, w):
    return pl.pallas_call(
        _mm_body,
        grid=(NN // _BN,),
        in_specs=[
            pl.BlockSpec((_BN, DD), lambda i: (i, 0)),
            pl.BlockSpec((DD, DD), lambda i: (0, 0)),
        ],
        out_specs=pl.BlockSpec((_BN, DD), lambda i: (i, 0)),
        out_shape=jax.ShapeDtypeStruct((NN, DD), jnp.float32),
    )(x, w)


def _scale_body(degp_ref, h_ref, dis_ref, g_ref):
    deg = degp_ref[0, :, :16] + degp_ref[1, :, :16] + 1.0
    dis = lax.rsqrt(deg)
    dis_ref[...] = dis
    g_ref[...] = dis[:, :1] * h_ref[...]


def _dis_and_scale(degp, h):
    return pl.pallas_call(
        _scale_body,
        grid=(NN // _BN,),
        in_specs=[
            pl.BlockSpec((NC, _BN, DD), lambda i: (0, i, 0)),
            pl.BlockSpec((_BN, DD), lambda i: (i, 0)),
        ],
        out_specs=[
            pl.BlockSpec((_BN, 16), lambda i: (i, 0)),
            pl.BlockSpec((_BN, DD), lambda i: (i, 0)),
        ],
        out_shape=[
            jax.ShapeDtypeStruct((NN, 16), jnp.float32),
            jax.ShapeDtypeStruct((NN, DD), jnp.float32),
        ],
    )(degp, h)


def _mid_body(sp_ref, g_ref, dis_ref, w_ref, o_ref):
    dis = dis_ref[...][:, :1]
    z = jnp.maximum(dis * (sp_ref[0] + sp_ref[1] + g_ref[...]), 0.0)
    o_ref[...] = dis * jnp.dot(z, w_ref[...],
                               preferred_element_type=jnp.float32)


def _mid_layer(sp, g, dis, w):
    return pl.pallas_call(
        _mid_body,
        grid=(NN // _BN,),
        in_specs=[
            pl.BlockSpec((NC, _BN, DD), lambda i: (0, i, 0)),
            pl.BlockSpec((_BN, DD), lambda i: (i, 0)),
            pl.BlockSpec((_BN, 16), lambda i: (i, 0)),
            pl.BlockSpec((DD, DD), lambda i: (0, 0)),
        ],
        out_specs=pl.BlockSpec((_BN, DD), lambda i: (i, 0)),
        out_shape=jax.ShapeDtypeStruct((NN, DD), jnp.float32),
    )(sp, g, dis, w)


def _final_body(sp_ref, g_ref, dis_ref, o_ref):
    dis = dis_ref[...][:, :1]
    o_ref[...] = dis * (sp_ref[0] + sp_ref[1] + g_ref[...])


def _final_layer(sp, g, dis):
    return pl.pallas_call(
        _final_body,
        grid=(NN // _BN,),
        in_specs=[
            pl.BlockSpec((NC, _BN, DD), lambda i: (0, i, 0)),
            pl.BlockSpec((_BN, DD), lambda i: (i, 0)),
            pl.BlockSpec((_BN, 16), lambda i: (i, 0)),
        ],
        out_specs=pl.BlockSpec((_BN, DD), lambda i: (i, 0)),
        out_shape=jax.ShapeDtypeStruct((NN, DD), jnp.float32),
    )(sp, g, dis)


def kernel(x, edge_index, W1, W2):
    src = edge_index[0]
    dst = edge_index[1]

    degp = _deg_partials(dst)
    h1 = _matmul(x, W1)
    dis, g1 = _dis_and_scale(degp, h1)
    s1p = _agg_partials(g1, src, dst)
    g2 = _mid_layer(s1p, g1, dis, W2)
    s2p = _agg_partials(g2, src, dst)
    out = _final_layer(s2p, g2, dis)
    return out


# speedup vs baseline: 24.4321x; 1.0043x over previous
import functools

import jax
import jax.numpy as jnp
from jax import lax
from jax.experimental import pallas as pl
from jax.experimental.pallas import tpu as pltpu
from jax.experimental.pallas import tpu_sc as plsc

NN = 10000
NP = 10240
EE = 320000
DD = 128
NC = 2
NS = 16
NW = NC * NS
ET = EE // NW
C = 80
NCHUNK = ET // C
RT = NP // NS
ZR = 64
DW = 32

_mesh = plsc.VectorSubcoreMesh(core_axis_name="c", subcore_axis_name="s")


def _fill(ref, rows, width, value):
    @pl.loop(0, rows)
    def _(i):
        @pl.loop(0, width // 16)
        def _(j):
            ref[i, pl.ds(j * 16, 16)] = jnp.full((16,), value, jnp.float32)


def _zero_acc(zer_v, acc_sh, sid, width=DD):
    _fill(zer_v, ZR, width, 0.0)

    @pl.loop(0, RT // ZR)
    def _(k):
        pltpu.sync_copy(zer_v, acc_sh.at[pl.ds(sid * RT + k * ZR, ZR)])


def _readback(acc_sh, out_hbm, cid, sid):
    pltpu.sync_copy(acc_sh.at[pl.ds(sid * RT, RT)],
                    out_hbm.at[cid, pl.ds(sid * RT, RT)])


@functools.partial(
    pl.kernel,
    out_type=jax.ShapeDtypeStruct((NC, NP, DW), jnp.float32),
    mesh=_mesh,
    scratch_types=[
        pltpu.VMEM((C,), jnp.int32),
        pltpu.VMEM((C,), jnp.int32),
        pltpu.VMEM((C, DW), jnp.float32),
        pltpu.VMEM((ZR, DW), jnp.float32),
        pltpu.VMEM_SHARED((NP, DW), jnp.float32),
        pltpu.SemaphoreType.DMA,
        pltpu.SemaphoreType.DMA,
    ],
)
def _deg_partials(dst_hbm, out_hbm, dia, dib, ones_v, zer_v, acc_sh,
                  sem_a, sem_b):
    cid = lax.axis_index("c")
    sid = lax.axis_index("s")
    wid = sid * NC + cid
    base = wid * ET

    _fill(ones_v, C, DW, 1.0)
    _zero_acc(zer_v, acc_sh, sid, DW)
    plsc.subcore_barrier()

    pltpu.sync_copy(dst_hbm.at[pl.ds(base, C)], dia)
    pltpu.async_copy(ones_v, acc_sh.at[dia], sem_a, add=True)

    @pl.loop(0, NCHUNK // 2)
    def _(k):
        ci = 2 * k
        pltpu.sync_copy(dst_hbm.at[pl.ds(base + (ci + 1) * C, C)], dib)
        pltpu.async_copy(ones_v, acc_sh.at[dib], sem_b, add=True)
        pltpu.make_async_copy(ones_v, acc_sh.at[dia], sem_a).wait()
        pltpu.sync_copy(dst_hbm.at[pl.ds(base + (ci + 2) * C, C)], dia)
        pltpu.async_copy(ones_v, acc_sh.at[dia], sem_a, add=True)
        pltpu.make_async_copy(ones_v, acc_sh.at[dib], sem_b).wait()

    pltpu.make_async_copy(ones_v, acc_sh.at[dia], sem_a).wait()
    plsc.subcore_barrier()
    _readback(acc_sh, out_hbm, cid, sid)


@functools.partial(
    pl.kernel,
    out_type=jax.ShapeDtypeStruct((NC, NP, DD), jnp.float32),
    mesh=_mesh,
    scratch_types=[
        pltpu.VMEM((C,), jnp.int32),
        pltpu.VMEM((C,), jnp.int32),
        pltpu.VMEM((C,), jnp.int32),
        pltpu.VMEM((C,), jnp.int32),
        pltpu.VMEM((C,), jnp.int32),
        pltpu.VMEM((C,), jnp.int32),
        pltpu.VMEM((C, DD), jnp.float32),
        pltpu.VMEM((C, DD), jnp.float32),
        pltpu.VMEM((C, DD), jnp.float32),
        pltpu.VMEM((ZR, DD), jnp.float32),
        pltpu.VMEM_SHARED((NP, DD), jnp.float32),
        pltpu.SemaphoreType.DMA,
        pltpu.SemaphoreType.DMA,
        pltpu.SemaphoreType.DMA,
        pltpu.SemaphoreType.DMA,
        pltpu.SemaphoreType.DMA,
        pltpu.SemaphoreType.DMA,
    ],
)
def _agg_partials(g_hbm, src_hbm, dst_hbm, out_hbm,
                  si0, si1, si2, di0, di1, di2, r0, r1, r2, zer_v, acc_sh,
                  g0, g1, g2, s0, s1, s2):
    cid = lax.axis_index("c")
    sid = lax.axis_index("s")
    wid = sid * NC + cid
    base = wid * ET
    si = (si0, si1, si2)
    di = (di0, di1, di2)
    rows = (r0, r1, r2)
    gsem = (g0, g1, g2)
    ssem = (s0, s1, s2)

    _zero_acc(zer_v, acc_sh, sid)
    plsc.subcore_barrier()

    def load_idx(j, ci):
        pltpu.sync_copy(src_hbm.at[pl.ds(base + ci * C, C)], si[j])
        pltpu.sync_copy(dst_hbm.at[pl.ds(base + ci * C, C)], di[j])

    def start_gather(j):
        pltpu.async_copy(g_hbm.at[si[j]], rows[j], gsem[j])

    def wait_gather(j):
        pltpu.make_async_copy(g_hbm.at[si[j]], rows[j], gsem[j]).wait()

    def start_scatter(j):
        pltpu.async_copy(rows[j], acc_sh.at[di[j]], ssem[j], add=True)

    def wait_scatter(j):
        pltpu.make_async_copy(rows[j], acc_sh.at[di[j]], ssem[j]).wait()

    for j in (0, 1):
        load_idx(j, j)
        start_gather(j)

    @pl.loop(0, NCHUNK // 3)
    def _(k):
        for j in range(3):
            ci = 3 * k + j
            jb = (j + 2) % 3
            wait_gather(j)
            start_scatter(j)
            if j == 0:
                @pl.when(k >= 1)
                def _():
                    wait_scatter(jb)
            else:
                wait_scatter(jb)
            load_idx(jb, ci + 2)
            start_gather(jb)

    wait_gather(0)
    start_scatter(0)
    wait_gather(1)
    start_scatter(1)
    wait_scatter(2)
    wait_scatter(0)
    wait_scatter(1)
    plsc.subcore_barrier()
    _readback(acc_sh, out_hbm, cid, sid)


_BN = 10000


def _mm_body(x_ref, w_ref, o_ref):
    o_ref[...] = jnp.dot(x_ref[...], w_ref[...],
                         preferred_element_type=jnp.float32)


def _matmul(x, w):
    return pl.pallas_call(
        _mm_body,
        grid=(NN // _BN,),
        in_specs=[
            pl.BlockSpec((_BN, DD), lambda i: (i, 0)),
            pl.BlockSpec((DD, DD), lambda i: (0, 0)),
        ],
        out_specs=pl.BlockSpec((_BN, DD), lambda i: (i, 0)),
        out_shape=jax.ShapeDtypeStruct((NN, DD), jnp.float32),
    )(x, w)


def _scale_body(degp_ref, h_ref, dis_ref, g_ref):
    deg = degp_ref[0, :, :16] + degp_ref[1, :, :16] + 1.0
    dis = lax.rsqrt(deg)
    dis_ref[...] = dis
    g_ref[...] = dis[:, :1] * h_ref[...]


def _dis_and_scale(degp, h):
    return pl.pallas_call(
        _scale_body,
        grid=(NN // _BN,),
        in_specs=[
            pl.BlockSpec((NC, _BN, DW), lambda i: (0, i, 0)),
            pl.BlockSpec((_BN, DD), lambda i: (i, 0)),
        ],
        out_specs=[
            pl.BlockSpec((_BN, 16), lambda i: (i, 0)),
            pl.BlockSpec((_BN, DD), lambda i: (i, 0)),
        ],
        out_shape=[
            jax.ShapeDtypeStruct((NN, 16), jnp.float32),
            jax.ShapeDtypeStruct((NN, DD), jnp.float32),
        ],
    )(degp, h)


def _mid_body(sp_ref, g_ref, dis_ref, w_ref, o_ref):
    dis = dis_ref[...][:, :1]
    z = jnp.maximum(dis * (sp_ref[0] + sp_ref[1] + g_ref[...]), 0.0)
    o_ref[...] = dis * jnp.dot(z, w_ref[...],
                               preferred_element_type=jnp.float32)


def _mid_layer(sp, g, dis, w):
    return pl.pallas_call(
        _mid_body,
        grid=(NN // _BN,),
        in_specs=[
            pl.BlockSpec((NC, _BN, DD), lambda i: (0, i, 0)),
            pl.BlockSpec((_BN, DD), lambda i: (i, 0)),
            pl.BlockSpec((_BN, 16), lambda i: (i, 0)),
            pl.BlockSpec((DD, DD), lambda i: (0, 0)),
        ],
        out_specs=pl.BlockSpec((_BN, DD), lambda i: (i, 0)),
        out_shape=jax.ShapeDtypeStruct((NN, DD), jnp.float32),
    )(sp, g, dis, w)


def _final_body(sp_ref, g_ref, dis_ref, o_ref):
    dis = dis_ref[...][:, :1]
    o_ref[...] = dis * (sp_ref[0] + sp_ref[1] + g_ref[...])


def _final_layer(sp, g, dis):
    return pl.pallas_call(
        _final_body,
        grid=(NN // _BN,),
        in_specs=[
            pl.BlockSpec((NC, _BN, DD), lambda i: (0, i, 0)),
            pl.BlockSpec((_BN, DD), lambda i: (i, 0)),
            pl.BlockSpec((_BN, 16), lambda i: (i, 0)),
        ],
        out_specs=pl.BlockSpec((_BN, DD), lambda i: (i, 0)),
        out_shape=jax.ShapeDtypeStruct((NN, DD), jnp.float32),
    )(sp, g, dis)


def kernel(x, edge_index, W1, W2):
    src = edge_index[0]
    dst = edge_index[1]

    degp = _deg_partials(dst)
    h1 = _matmul(x, W1)
    dis, g1 = _dis_and_scale(degp, h1)
    s1p = _agg_partials(g1, src, dst)
    g2 = _mid_layer(s1p, g1, dis, W2)
    s2p = _agg_partials(g2, src, dst)
    out = _final_layer(s2p, g2, dis)
    return out


# speedup vs baseline: 27.9280x; 1.1431x over previous
import dataclasses
import functools

import jax
import jax.numpy as jnp
from jax import lax
from jax.experimental import pallas as pl
from jax.experimental.pallas import tpu as pltpu
from jax.experimental.pallas import tpu_sc as plsc

NN = 10000
NP = 10240
EE = 320000
DD = 128
NC = 2
NS = 16
NW = NC * NS
ET = EE // NW
C = 80
NCHUNK = ET // C
RT = NP // NS
ZR = 64

_mesh = plsc.VectorSubcoreMesh(core_axis_name="c", subcore_axis_name="s")


def _fill(ref, rows, width, value):
    @pl.loop(0, rows)
    def _(i):
        @pl.loop(0, width // 16)
        def _(j):
            ref[i, pl.ds(j * 16, 16)] = jnp.full((16,), value, jnp.float32)


def _zero_acc(zer_v, acc_sh, sid, width=DD):
    _fill(zer_v, ZR, width, 0.0)

    @pl.loop(0, RT // ZR)
    def _(k):
        pltpu.sync_copy(zer_v, acc_sh.at[pl.ds(sid * RT + k * ZR, ZR)])


def _readback(acc_sh, out_hbm, cid, sid):
    pltpu.sync_copy(acc_sh.at[pl.ds(sid * RT, RT)],
                    out_hbm.at[cid, pl.ds(sid * RT, RT)])


_cp = pltpu.CompilerParams()
if "needs_layout_passes" in pltpu.CompilerParams.__dataclass_fields__:
    _cp = dataclasses.replace(_cp, needs_layout_passes=False)


@functools.partial(
    pl.kernel,
    out_type=jax.ShapeDtypeStruct((NW * NP,), jnp.float32),
    mesh=_mesh,
    compiler_params=_cp,
    scratch_types=[
        pltpu.VMEM((NP,), jnp.float32),
        pltpu.VMEM((ET,), jnp.int32),
    ],
)
def _deg_partials(dst_hbm, out_hbm, acc_v, didx_v):
    cid = lax.axis_index("c")
    sid = lax.axis_index("s")
    wid = sid * NC + cid

    @pl.loop(0, NP // 16)
    def _(i):
        acc_v[pl.ds(i * 16, 16)] = jnp.zeros((16,), jnp.float32)

    pltpu.sync_copy(dst_hbm.at[pl.ds(wid * ET, ET)], didx_v)
    ones16 = jnp.ones((16,), jnp.float32)

    @pl.loop(0, ET // 16)
    def _(i):
        idx = didx_v[pl.ds(i * 16, 16)]
        plsc.addupdate_scatter(acc_v, [idx], ones16)

    pltpu.sync_copy(acc_v, out_hbm.at[pl.ds(wid * NP, NP)])


@functools.partial(
    pl.kernel,
    out_type=jax.ShapeDtypeStruct((NC, NP, DD), jnp.float32),
    mesh=_mesh,
    scratch_types=[
        pltpu.VMEM((C,), jnp.int32),
        pltpu.VMEM((C,), jnp.int32),
        pltpu.VMEM((C,), jnp.int32),
        pltpu.VMEM((C,), jnp.int32),
        pltpu.VMEM((C,), jnp.int32),
        pltpu.VMEM((C,), jnp.int32),
        pltpu.VMEM((C, DD), jnp.float32),
        pltpu.VMEM((C, DD), jnp.float32),
        pltpu.VMEM((C, DD), jnp.float32),
        pltpu.VMEM((ZR, DD), jnp.float32),
        pltpu.VMEM_SHARED((NP, DD), jnp.float32),
        pltpu.SemaphoreType.DMA,
        pltpu.SemaphoreType.DMA,
        pltpu.SemaphoreType.DMA,
        pltpu.SemaphoreType.DMA,
        pltpu.SemaphoreType.DMA,
        pltpu.SemaphoreType.DMA,
    ],
)
def _agg_partials(g_hbm, src_hbm, dst_hbm, out_hbm,
                  si0, si1, si2, di0, di1, di2, r0, r1, r2, zer_v, acc_sh,
                  g0, g1, g2, s0, s1, s2):
    cid = lax.axis_index("c")
    sid = lax.axis_index("s")
    wid = sid * NC + cid
    base = wid * ET
    si = (si0, si1, si2)
    di = (di0, di1, di2)
    rows = (r0, r1, r2)
    gsem = (g0, g1, g2)
    ssem = (s0, s1, s2)

    _zero_acc(zer_v, acc_sh, sid)
    plsc.subcore_barrier()

    def load_idx(j, ci):
        pltpu.sync_copy(src_hbm.at[pl.ds(base + ci * C, C)], si[j])
        pltpu.sync_copy(dst_hbm.at[pl.ds(base + ci * C, C)], di[j])

    def start_gather(j):
        pltpu.async_copy(g_hbm.at[si[j]], rows[j], gsem[j])

    def wait_gather(j):
        pltpu.make_async_copy(g_hbm.at[si[j]], rows[j], gsem[j]).wait()

    def start_scatter(j):
        pltpu.async_copy(rows[j], acc_sh.at[di[j]], ssem[j], add=True)

    def wait_scatter(j):
        pltpu.make_async_copy(rows[j], acc_sh.at[di[j]], ssem[j]).wait()

    for j in (0, 1):
        load_idx(j, j)
        start_gather(j)

    @pl.loop(0, NCHUNK // 3)
    def _(k):
        for j in range(3):
            ci = 3 * k + j
            jb = (j + 2) % 3
            wait_gather(j)
            start_scatter(j)
            if j == 0:
                @pl.when(k >= 1)
                def _():
                    wait_scatter(jb)
            else:
                wait_scatter(jb)
            load_idx(jb, ci + 2)
            start_gather(jb)

    wait_gather(0)
    start_scatter(0)
    wait_gather(1)
    start_scatter(1)
    wait_scatter(2)
    wait_scatter(0)
    wait_scatter(1)
    plsc.subcore_barrier()
    _readback(acc_sh, out_hbm, cid, sid)


_BN = 10000


def _mm_body(x_ref, w_ref, o_ref):
    o_ref[...] = jnp.dot(x_ref[...], w_ref[...],
                         preferred_element_type=jnp.float32)


def _matmul(x, w):
    return pl.pallas_call(
        _mm_body,
        grid=(NN // _BN,),
        in_specs=[
            pl.BlockSpec((_BN, DD), lambda i: (i, 0)),
            pl.BlockSpec((DD, DD), lambda i: (0, 0)),
        ],
        out_specs=pl.BlockSpec((_BN, DD), lambda i: (i, 0)),
        out_shape=jax.ShapeDtypeStruct((NN, DD), jnp.float32),
    )(x, w)


def _scale_body(degp_ref, h_ref, dis_ref, g_ref):
    ones_w = jnp.ones((NW, 1), jnp.float32)
    deg = lax.dot_general(degp_ref[...], ones_w, (((0,), (0,)), ((), ())),
                          precision=lax.Precision.HIGHEST,
                          preferred_element_type=jnp.float32)
    dis_full = lax.rsqrt(deg + 1.0)
    dis = dis_full[:NN]
    dis_ref[...] = dis
    g_ref[...] = dis * h_ref[...]


def _dis_and_scale(degp, h):
    return pl.pallas_call(
        _scale_body,
        grid=(NN // _BN,),
        in_specs=[
            pl.BlockSpec((NW, NP), lambda i: (0, 0)),
            pl.BlockSpec((_BN, DD), lambda i: (i, 0)),
        ],
        out_specs=[
            pl.BlockSpec((_BN, 1), lambda i: (i, 0)),
            pl.BlockSpec((_BN, DD), lambda i: (i, 0)),
        ],
        out_shape=[
            jax.ShapeDtypeStruct((NN, 1), jnp.float32),
            jax.ShapeDtypeStruct((NN, DD), jnp.float32),
        ],
    )(degp, h)


def _mid_body(sp_ref, g_ref, dis_ref, w_ref, o_ref):
    dis = dis_ref[...]
    z = jnp.maximum(dis * (sp_ref[0] + sp_ref[1] + g_ref[...]), 0.0)
    o_ref[...] = dis * jnp.dot(z, w_ref[...],
                               preferred_element_type=jnp.float32)


def _mid_layer(sp, g, dis, w):
    return pl.pallas_call(
        _mid_body,
        grid=(NN // _BN,),
        in_specs=[
            pl.BlockSpec((NC, _BN, DD), lambda i: (0, i, 0)),
            pl.BlockSpec((_BN, DD), lambda i: (i, 0)),
            pl.BlockSpec((_BN, 1), lambda i: (i, 0)),
            pl.BlockSpec((DD, DD), lambda i: (0, 0)),
        ],
        out_specs=pl.BlockSpec((_BN, DD), lambda i: (i, 0)),
        out_shape=jax.ShapeDtypeStruct((NN, DD), jnp.float32),
    )(sp, g, dis, w)


def _final_body(sp_ref, g_ref, dis_ref, o_ref):
    dis = dis_ref[...]
    o_ref[...] = dis * (sp_ref[0] + sp_ref[1] + g_ref[...])


def _final_layer(sp, g, dis):
    return pl.pallas_call(
        _final_body,
        grid=(NN // _BN,),
        in_specs=[
            pl.BlockSpec((NC, _BN, DD), lambda i: (0, i, 0)),
            pl.BlockSpec((_BN, DD), lambda i: (i, 0)),
            pl.BlockSpec((_BN, 1), lambda i: (i, 0)),
        ],
        out_specs=pl.BlockSpec((_BN, DD), lambda i: (i, 0)),
        out_shape=jax.ShapeDtypeStruct((NN, DD), jnp.float32),
    )(sp, g, dis)


def kernel(x, edge_index, W1, W2):
    src = edge_index[0]
    dst = edge_index[1]

    degp = _deg_partials(dst).reshape(NW, NP)
    h1 = _matmul(x, W1)
    dis, g1 = _dis_and_scale(degp, h1)
    s1p = _agg_partials(g1, src, dst)
    g2 = _mid_layer(s1p, g1, dis, W2)
    s2p = _agg_partials(g2, src, dst)
    out = _final_layer(s2p, g2, dis)
    return out


# speedup vs baseline: 28.4393x; 1.0183x over previous
import dataclasses
import functools

import jax
import jax.numpy as jnp
from jax import lax
from jax.experimental import pallas as pl
from jax.experimental.pallas import tpu as pltpu
from jax.experimental.pallas import tpu_sc as plsc

NN = 10000
NP = 10240
EE = 320000
DD = 128
NC = 2
NS = 16
NW = NC * NS
ET = EE // NW
C = 80
NCHUNK = ET // C
RT = NP // NS
ZR = 64

_mesh = plsc.VectorSubcoreMesh(core_axis_name="c", subcore_axis_name="s")


def _fill(ref, rows, width, value):
    @pl.loop(0, rows)
    def _(i):
        @pl.loop(0, width // 16)
        def _(j):
            ref[i, pl.ds(j * 16, 16)] = jnp.full((16,), value, jnp.float32)


def _zero_acc_start(zer_v, acc_sh, sid, zsem):
    _fill(zer_v, ZR, DD, 0.0)
    for k in range(RT // ZR):
        pltpu.async_copy(zer_v, acc_sh.at[pl.ds(sid * RT + k * ZR, ZR)], zsem)


def _zero_acc_wait(zer_v, acc_sh, sid, zsem):
    for k in range(RT // ZR):
        pltpu.make_async_copy(
            zer_v, acc_sh.at[pl.ds(sid * RT + k * ZR, ZR)], zsem).wait()


def _readback(acc_sh, out_hbm, cid, sid):
    pltpu.sync_copy(acc_sh.at[pl.ds(sid * RT, RT)],
                    out_hbm.at[cid, pl.ds(sid * RT, RT)])


_cp = pltpu.CompilerParams()
if "needs_layout_passes" in pltpu.CompilerParams.__dataclass_fields__:
    _cp = dataclasses.replace(_cp, needs_layout_passes=False)


@functools.partial(
    pl.kernel,
    out_type=jax.ShapeDtypeStruct((NW * NP,), jnp.float32),
    mesh=_mesh,
    compiler_params=_cp,
    scratch_types=[
        pltpu.VMEM((NP,), jnp.float32),
        pltpu.VMEM((ET,), jnp.int32),
        pltpu.SemaphoreType.DMA,
    ],
)
def _deg_partials(dst_hbm, out_hbm, acc_v, didx_v, isem):
    cid = lax.axis_index("c")
    sid = lax.axis_index("s")
    wid = sid * NC + cid

    cp = pltpu.async_copy(dst_hbm.at[pl.ds(wid * ET, ET)], didx_v, isem)

    @pl.loop(0, NP // 16)
    def _(i):
        acc_v[pl.ds(i * 16, 16)] = jnp.zeros((16,), jnp.float32)

    cp.wait()
    ones16 = jnp.ones((16,), jnp.float32)

    @pl.loop(0, ET // 80)
    def _(i):
        for u in range(5):
            idx = didx_v[pl.ds(i * 80 + u * 16, 16)]
            plsc.addupdate_scatter(acc_v, [idx], ones16)

    pltpu.sync_copy(acc_v, out_hbm.at[pl.ds(wid * NP, NP)])


@functools.partial(
    pl.kernel,
    out_type=jax.ShapeDtypeStruct((NC, NP, DD), jnp.float32),
    mesh=_mesh,
    scratch_types=[
        pltpu.VMEM((C,), jnp.int32),
        pltpu.VMEM((C,), jnp.int32),
        pltpu.VMEM((C,), jnp.int32),
        pltpu.VMEM((C,), jnp.int32),
        pltpu.VMEM((C,), jnp.int32),
        pltpu.VMEM((C,), jnp.int32),
        pltpu.VMEM((C, DD), jnp.float32),
        pltpu.VMEM((C, DD), jnp.float32),
        pltpu.VMEM((C, DD), jnp.float32),
        pltpu.VMEM((ZR, DD), jnp.float32),
        pltpu.VMEM_SHARED((NP, DD), jnp.float32),
        pltpu.SemaphoreType.DMA,
        pltpu.SemaphoreType.DMA,
        pltpu.SemaphoreType.DMA,
        pltpu.SemaphoreType.DMA,
        pltpu.SemaphoreType.DMA,
        pltpu.SemaphoreType.DMA,
        pltpu.SemaphoreType.DMA,
    ],
)
def _agg_partials(g_hbm, src_hbm, dst_hbm, out_hbm,
                  si0, si1, si2, di0, di1, di2, r0, r1, r2, zer_v, acc_sh,
                  g0, g1, g2, s0, s1, s2, zsem):
    cid = lax.axis_index("c")
    sid = lax.axis_index("s")
    wid = sid * NC + cid
    base = wid * ET
    si = (si0, si1, si2)
    di = (di0, di1, di2)
    rows = (r0, r1, r2)
    gsem = (g0, g1, g2)
    ssem = (s0, s1, s2)

    _zero_acc_start(zer_v, acc_sh, sid, zsem)

    def load_idx(j, ci):
        pltpu.sync_copy(src_hbm.at[pl.ds(base + ci * C, C)], si[j])
        pltpu.sync_copy(dst_hbm.at[pl.ds(base + ci * C, C)], di[j])

    def start_gather(j):
        pltpu.async_copy(g_hbm.at[si[j]], rows[j], gsem[j])

    def wait_gather(j):
        pltpu.make_async_copy(g_hbm.at[si[j]], rows[j], gsem[j]).wait()

    def start_scatter(j):
        pltpu.async_copy(rows[j], acc_sh.at[di[j]], ssem[j], add=True)

    def wait_scatter(j):
        pltpu.make_async_copy(rows[j], acc_sh.at[di[j]], ssem[j]).wait()

    for j in (0, 1):
        load_idx(j, j)
        start_gather(j)
    _zero_acc_wait(zer_v, acc_sh, sid, zsem)
    plsc.subcore_barrier()

    @pl.loop(0, NCHUNK // 3)
    def _(k):
        for j in range(3):
            ci = 3 * k + j
            jb = (j + 2) % 3
            wait_gather(j)
            start_scatter(j)
            if j == 0:
                @pl.when(k >= 1)
                def _():
                    wait_scatter(jb)
            else:
                wait_scatter(jb)
            load_idx(jb, ci + 2)
            start_gather(jb)

    wait_gather(0)
    start_scatter(0)
    wait_gather(1)
    start_scatter(1)
    wait_scatter(2)
    wait_scatter(0)
    wait_scatter(1)
    plsc.subcore_barrier()
    _readback(acc_sh, out_hbm, cid, sid)


_BN = 10000


def _mm_body(x_ref, w_ref, o_ref):
    o_ref[...] = jnp.dot(x_ref[...], w_ref[...],
                         preferred_element_type=jnp.float32)


def _matmul(x, w):
    return pl.pallas_call(
        _mm_body,
        grid=(NN // _BN,),
        in_specs=[
            pl.BlockSpec((_BN, DD), lambda i: (i, 0)),
            pl.BlockSpec((DD, DD), lambda i: (0, 0)),
        ],
        out_specs=pl.BlockSpec((_BN, DD), lambda i: (i, 0)),
        out_shape=jax.ShapeDtypeStruct((NN, DD), jnp.float32),
    )(x, w)


def _scale_body(degp_ref, h_ref, dis_ref, g_ref):
    ones_w = jnp.ones((NW, 1), jnp.float32)
    deg = lax.dot_general(degp_ref[...], ones_w, (((0,), (0,)), ((), ())),
                          precision=lax.Precision.HIGHEST,
                          preferred_element_type=jnp.float32)
    dis_full = lax.rsqrt(deg + 1.0)
    dis = dis_full[:NN]
    dis_ref[...] = dis
    g_ref[...] = dis * h_ref[...]


def _dis_and_scale(degp, h):
    return pl.pallas_call(
        _scale_body,
        grid=(NN // _BN,),
        in_specs=[
            pl.BlockSpec((NW, NP), lambda i: (0, 0)),
            pl.BlockSpec((_BN, DD), lambda i: (i, 0)),
        ],
        out_specs=[
            pl.BlockSpec((_BN, 1), lambda i: (i, 0)),
            pl.BlockSpec((_BN, DD), lambda i: (i, 0)),
        ],
        out_shape=[
            jax.ShapeDtypeStruct((NN, 1), jnp.float32),
            jax.ShapeDtypeStruct((NN, DD), jnp.float32),
        ],
    )(degp, h)


def _mid_body(sp_ref, g_ref, dis_ref, w_ref, o_ref):
    dis = dis_ref[...]
    z = jnp.maximum(dis * (sp_ref[0] + sp_ref[1] + g_ref[...]), 0.0)
    o_ref[...] = dis * jnp.dot(z, w_ref[...],
                               preferred_element_type=jnp.float32)


def _mid_layer(sp, g, dis, w):
    return pl.pallas_call(
        _mid_body,
        grid=(NN // _BN,),
        in_specs=[
            pl.BlockSpec((NC, _BN, DD), lambda i: (0, i, 0)),
            pl.BlockSpec((_BN, DD), lambda i: (i, 0)),
            pl.BlockSpec((_BN, 1), lambda i: (i, 0)),
            pl.BlockSpec((DD, DD), lambda i: (0, 0)),
        ],
        out_specs=pl.BlockSpec((_BN, DD), lambda i: (i, 0)),
        out_shape=jax.ShapeDtypeStruct((NN, DD), jnp.float32),
    )(sp, g, dis, w)


def _final_body(sp_ref, g_ref, dis_ref, o_ref):
    dis = dis_ref[...]
    o_ref[...] = dis * (sp_ref[0] + sp_ref[1] + g_ref[...])


def _final_layer(sp, g, dis):
    return pl.pallas_call(
        _final_body,
        grid=(NN // _BN,),
        in_specs=[
            pl.BlockSpec((NC, _BN, DD), lambda i: (0, i, 0)),
            pl.BlockSpec((_BN, DD), lambda i: (i, 0)),
            pl.BlockSpec((_BN, 1), lambda i: (i, 0)),
        ],
        out_specs=pl.BlockSpec((_BN, DD), lambda i: (i, 0)),
        out_shape=jax.ShapeDtypeStruct((NN, DD), jnp.float32),
    )(sp, g, dis)


def kernel(x, edge_index, W1, W2):
    src = edge_index[0]
    dst = edge_index[1]

    degp = _deg_partials(dst).reshape(NW, NP)
    h1 = _matmul(x, W1)
    dis, g1 = _dis_and_scale(degp, h1)
    s1p = _agg_partials(g1, src, dst)
    g2 = _mid_layer(s1p, g1, dis, W2)
    s2p = _agg_partials(g2, src, dst)
    out = _final_layer(s2p, g2, dis)
    return out


# speedup vs baseline: 30.7950x; 1.0828x over previous
import dataclasses
import functools

import jax
import jax.numpy as jnp
from jax import lax
from jax.experimental import pallas as pl
from jax.experimental.pallas import tpu as pltpu
from jax.experimental.pallas import tpu_sc as plsc

NN = 10000
NP = 10240
EE = 320000
DD = 128
NC = 2
NS = 16
NW = NC * NS
ET = EE // NW
C = 80
NCHUNK = ET // C
RT = NP // NS
ZR = 32

_mesh = plsc.VectorSubcoreMesh(core_axis_name="c", subcore_axis_name="s")


def _fill(ref, rows, width, value):
    @pl.loop(0, rows)
    def _(i):
        @pl.loop(0, width // 16)
        def _(j):
            ref[i, pl.ds(j * 16, 16)] = jnp.full((16,), value, jnp.float32)


def _zero_acc_start(zer_v, acc_sh, sid, zsem):
    _fill(zer_v, ZR, DD, 0.0)
    for k in range(RT // ZR):
        pltpu.async_copy(zer_v, acc_sh.at[pl.ds(sid * RT + k * ZR, ZR)], zsem)


def _zero_acc_wait(zer_v, acc_sh, sid, zsem):
    for k in range(RT // ZR):
        pltpu.make_async_copy(
            zer_v, acc_sh.at[pl.ds(sid * RT + k * ZR, ZR)], zsem).wait()


def _readback(acc_sh, out_hbm, cid, sid):
    pltpu.sync_copy(acc_sh.at[pl.ds(sid * RT, RT)],
                    out_hbm.at[cid, pl.ds(sid * RT, RT)])


_cp = pltpu.CompilerParams()
if "needs_layout_passes" in pltpu.CompilerParams.__dataclass_fields__:
    _cp = dataclasses.replace(_cp, needs_layout_passes=False)


@functools.partial(
    pl.kernel,
    out_type=jax.ShapeDtypeStruct((NW * NP,), jnp.float32),
    mesh=_mesh,
    compiler_params=_cp,
    scratch_types=[
        pltpu.VMEM((NP,), jnp.float32),
        pltpu.VMEM((ET,), jnp.int32),
        pltpu.SemaphoreType.DMA,
    ],
)
def _deg_partials(dst_hbm, out_hbm, acc_v, didx_v, isem):
    cid = lax.axis_index("c")
    sid = lax.axis_index("s")
    wid = sid * NC + cid

    cp = pltpu.async_copy(dst_hbm.at[pl.ds(wid * ET, ET)], didx_v, isem)

    @pl.loop(0, NP // 16)
    def _(i):
        acc_v[pl.ds(i * 16, 16)] = jnp.zeros((16,), jnp.float32)

    cp.wait()
    ones16 = jnp.ones((16,), jnp.float32)

    @pl.loop(0, ET // 80)
    def _(i):
        for u in range(5):
            idx = didx_v[pl.ds(i * 80 + u * 16, 16)]
            plsc.addupdate_scatter(acc_v, [idx], ones16)

    pltpu.sync_copy(acc_v, out_hbm.at[pl.ds(wid * NP, NP)])


@functools.partial(
    pl.kernel,
    out_type=jax.ShapeDtypeStruct((NC, NP, DD), jnp.float32),
    mesh=_mesh,
    scratch_types=(
        [pltpu.VMEM((C,), jnp.int32) for _ in range(4)]
        + [pltpu.VMEM((C,), jnp.int32) for _ in range(4)]
        + [pltpu.VMEM((C, DD), jnp.float32) for _ in range(4)]
        + [pltpu.VMEM((ZR, DD), jnp.float32),
           pltpu.VMEM_SHARED((NP, DD), jnp.float32)]
        + [pltpu.SemaphoreType.DMA for _ in range(9)]
    ),
)
def _agg_partials(g_hbm, src_hbm, dst_hbm, out_hbm,
                  si0, si1, si2, si3, di0, di1, di2, di3,
                  r0, r1, r2, r3, zer_v, acc_sh,
                  gs0, gs1, gs2, gs3, ss0, ss1, ss2, ss3, zsem):
    cid = lax.axis_index("c")
    sid = lax.axis_index("s")
    wid = sid * NC + cid
    base = wid * ET
    si = (si0, si1, si2, si3)
    di = (di0, di1, di2, di3)
    rows = (r0, r1, r2, r3)
    gsem = (gs0, gs1, gs2, gs3)
    ssem = (ss0, ss1, ss2, ss3)

    _zero_acc_start(zer_v, acc_sh, sid, zsem)

    def load_idx(j, ci):
        pltpu.sync_copy(src_hbm.at[pl.ds(base + ci * C, C)], si[j])
        pltpu.sync_copy(dst_hbm.at[pl.ds(base + ci * C, C)], di[j])

    def start_gather(j):
        pltpu.async_copy(g_hbm.at[si[j]], rows[j], gsem[j])

    def wait_gather(j):
        pltpu.make_async_copy(g_hbm.at[si[j]], rows[j], gsem[j]).wait()

    def start_scatter(j):
        pltpu.async_copy(rows[j], acc_sh.at[di[j]], ssem[j], add=True)

    def wait_scatter(j):
        pltpu.make_async_copy(rows[j], acc_sh.at[di[j]], ssem[j]).wait()

    for j in (0, 1, 2):
        load_idx(j, j)
        start_gather(j)
    _zero_acc_wait(zer_v, acc_sh, sid, zsem)
    plsc.subcore_barrier()

    @pl.loop(0, 30)
    def _(k):
        for j in range(4):
            ci = 4 * k + j
            jn = (j + 3) % 4
            wait_gather(j)
            start_scatter(j)
            if j == 0:
                @pl.when(k >= 1)
                def _():
                    wait_scatter(jn)
            else:
                wait_scatter(jn)
            load_idx(jn, ci + 3)
            start_gather(jn)

    wait_gather(0)
    start_scatter(0)
    wait_scatter(3)
    load_idx(3, 123)
    start_gather(3)
    wait_gather(1)
    start_scatter(1)
    wait_scatter(0)
    load_idx(0, 124)
    start_gather(0)
    wait_gather(2)
    start_scatter(2)
    wait_scatter(1)
    wait_gather(3)
    start_scatter(3)
    wait_scatter(2)
    wait_gather(0)
    start_scatter(0)
    wait_scatter(3)
    wait_scatter(0)
    plsc.subcore_barrier()
    _readback(acc_sh, out_hbm, cid, sid)


_BN = 10000


def _mm_body(x_ref, w_ref, o_ref):
    o_ref[...] = jnp.dot(x_ref[...], w_ref[...],
                         preferred_element_type=jnp.float32)


def _matmul(x, w):
    return pl.pallas_call(
        _mm_body,
        grid=(NN // _BN,),
        in_specs=[
            pl.BlockSpec((_BN, DD), lambda i: (i, 0)),
            pl.BlockSpec((DD, DD), lambda i: (0, 0)),
        ],
        out_specs=pl.BlockSpec((_BN, DD), lambda i: (i, 0)),
        out_shape=jax.ShapeDtypeStruct((NN, DD), jnp.float32),
    )(x, w)


def _scale_body(degp_ref, h_ref, dis_ref, g_ref):
    ones_w = jnp.ones((NW, 1), jnp.float32)
    deg = lax.dot_general(degp_ref[...], ones_w, (((0,), (0,)), ((), ())),
                          precision=lax.Precision.HIGHEST,
                          preferred_element_type=jnp.float32)
    dis_full = lax.rsqrt(deg + 1.0)
    dis = dis_full[:NN]
    dis_ref[...] = dis
    g_ref[...] = dis * h_ref[...]


def _dis_and_scale(degp, h):
    return pl.pallas_call(
        _scale_body,
        grid=(NN // _BN,),
        in_specs=[
            pl.BlockSpec((NW, NP), lambda i: (0, 0)),
            pl.BlockSpec((_BN, DD), lambda i: (i, 0)),
        ],
        out_specs=[
            pl.BlockSpec((_BN, 1), lambda i: (i, 0)),
            pl.BlockSpec((_BN, DD), lambda i: (i, 0)),
        ],
        out_shape=[
            jax.ShapeDtypeStruct((NN, 1), jnp.float32),
            jax.ShapeDtypeStruct((NN, DD), jnp.float32),
        ],
    )(degp, h)


def _mid_body(sp_ref, g_ref, dis_ref, w_ref, o_ref):
    dis = dis_ref[...]
    z = jnp.maximum(dis * (sp_ref[0] + sp_ref[1] + g_ref[...]), 0.0)
    o_ref[...] = dis * jnp.dot(z, w_ref[...],
                               preferred_element_type=jnp.float32)


def _mid_layer(sp, g, dis, w):
    return pl.pallas_call(
        _mid_body,
        grid=(NN // _BN,),
        in_specs=[
            pl.BlockSpec((NC, _BN, DD), lambda i: (0, i, 0)),
            pl.BlockSpec((_BN, DD), lambda i: (i, 0)),
            pl.BlockSpec((_BN, 1), lambda i: (i, 0)),
            pl.BlockSpec((DD, DD), lambda i: (0, 0)),
        ],
        out_specs=pl.BlockSpec((_BN, DD), lambda i: (i, 0)),
        out_shape=jax.ShapeDtypeStruct((NN, DD), jnp.float32),
    )(sp, g, dis, w)


def _final_body(sp_ref, g_ref, dis_ref, o_ref):
    dis = dis_ref[...]
    o_ref[...] = dis * (sp_ref[0] + sp_ref[1] + g_ref[...])


def _final_layer(sp, g, dis):
    return pl.pallas_call(
        _final_body,
        grid=(NN // _BN,),
        in_specs=[
            pl.BlockSpec((NC, _BN, DD), lambda i: (0, i, 0)),
            pl.BlockSpec((_BN, DD), lambda i: (i, 0)),
            pl.BlockSpec((_BN, 1), lambda i: (i, 0)),
        ],
        out_specs=pl.BlockSpec((_BN, DD), lambda i: (i, 0)),
        out_shape=jax.ShapeDtypeStruct((NN, DD), jnp.float32),
    )(sp, g, dis)


def kernel(x, edge_index, W1, W2):
    src = edge_index[0]
    dst = edge_index[1]

    degp = _deg_partials(dst).reshape(NW, NP)
    h1 = _matmul(x, W1)
    dis, g1 = _dis_and_scale(degp, h1)
    s1p = _agg_partials(g1, src, dst)
    g2 = _mid_layer(s1p, g1, dis, W2)
    s2p = _agg_partials(g2, src, dst)
    out = _final_layer(s2p, g2, dis)
    return out
